# Initial kernel scaffold; baseline (speedup 1.0000x reference)
import jax
import jax.numpy as jnp
from jax.experimental import pallas as pl


def kernel(x, edge_index, W1, b1, W2, b2, W3, b3, W4, b4):
    raise NotImplementedError("write your pallas kernel here")


# speedup vs baseline: 63.3355x; 63.3355x over previous
import functools

import jax
import jax.numpy as jnp
from jax import lax
from jax.experimental import pallas as pl
from jax.experimental.pallas import tpu as pltpu
from jax.experimental.pallas import tpu_sc as plsc

N_NODES = 50000
N_PAD = 51200
N_EDGES = 3200000
LANE = 128
KROW = 8
E_PAD = 3211264
E_ROWS = E_PAD // LANE
NC, NS = 2, 16
RPT = 1600


def _mesh():
    return plsc.VectorSubcoreMesh(core_axis_name="c", subcore_axis_name="s",
                                  num_cores=NC, num_subcores=NS)


def _make_deg(n_pad, e_rows, interpret=False):
    rows_per_tile = e_rows // (NC * NS)
    niter = rows_per_tile // KROW
    rpt = n_pad // NS

    @functools.partial(
        pl.kernel,
        out_type=(
            jax.ShapeDtypeStruct((n_pad, 16), jnp.float32),
            jax.ShapeDtypeStruct((n_pad, 16), jnp.float32),
        ),
        mesh=_mesh(),
        compiler_params=pltpu.CompilerParams(use_tc_tiling_on_sc=False),
        scratch_types=[
            pltpu.VMEM_SHARED((n_pad, 16), jnp.float32),
            pltpu.VMEM((KROW, LANE), jnp.int32),
            pltpu.VMEM((LANE, 16), jnp.float32),
        ],
        interpret=interpret,
    )
    def deg_kernel(dst_hbm, zeros_hbm, ones_hbm, o0, o1, acc_sh, didx, ones_v):
        cid = lax.axis_index("c")
        sid = lax.axis_index("s")
        r0 = sid * rpt
        pltpu.sync_copy(zeros_hbm.at[pl.ds(r0, rpt)], acc_sh.at[pl.ds(r0, rpt)])
        pltpu.sync_copy(ones_hbm, ones_v)
        plsc.subcore_barrier()
        base = (cid * NS + sid) * rows_per_tile

        def body(g, carry):
            row = base + g * KROW
            pltpu.sync_copy(dst_hbm.at[pl.ds(row, KROW)], didx)
            for j in range(KROW):
                pltpu.sync_copy(ones_v, acc_sh.at[didx.at[j]], add=True)
            return carry

        lax.fori_loop(0, niter, body, 0)
        plsc.subcore_barrier()

        @pl.when(cid == 0)
        def _():
            pltpu.sync_copy(acc_sh.at[pl.ds(r0, rpt)], o0.at[pl.ds(r0, rpt)])

        @pl.when(cid == 1)
        def _():
            pltpu.sync_copy(acc_sh.at[pl.ds(r0, rpt)], o1.at[pl.ds(r0, rpt)])

    return deg_kernel


def _make_prop(n_pad, e_rows, feat_split, interpret=False):
    nworkers = NS if feat_split else NC * NS
    rows_per_tile = e_rows // nworkers
    niter = rows_per_tile // KROW
    rpt = n_pad // NS

    def body_fn(t0, t1, src_hbm, dst_hbm, zeros_hbm, o0, o1,
                table_sh, acc_sh, sidx, didx, rows_v, sem):
        cid = lax.axis_index("c")
        sid = lax.axis_index("s")
        r0 = sid * rpt
        nd = pl.ds(r0, rpt)
        if feat_split:
            @pl.when(cid == 0)
            def _():
                pltpu.sync_copy(t0.at[nd], table_sh.at[nd])

            @pl.when(cid == 1)
            def _():
                pltpu.sync_copy(t1.at[nd], table_sh.at[nd])
        else:
            pltpu.sync_copy(t0.at[nd], table_sh.at[nd])
        pltpu.sync_copy(zeros_hbm.at[nd], acc_sh.at[nd])
        plsc.subcore_barrier()

        if feat_split:
            base = sid * rows_per_tile
        else:
            base = (cid * NS + sid) * rows_per_tile

        def body(g, carry):
            row = base + g * KROW
            pltpu.sync_copy(src_hbm.at[pl.ds(row, KROW)], sidx)
            pltpu.sync_copy(dst_hbm.at[pl.ds(row, KROW)], didx)
            for j in range(KROW):
                pltpu.async_copy(table_sh.at[sidx.at[j]], rows_v, sem).wait()
                pltpu.sync_copy(rows_v, acc_sh.at[didx.at[j]], add=True)
            return carry

        lax.fori_loop(0, niter, body, 0)
        plsc.subcore_barrier()

        @pl.when(cid == 0)
        def _():
            pltpu.sync_copy(acc_sh.at[nd], o0.at[nd])

        @pl.when(cid == 1)
        def _():
            pltpu.sync_copy(acc_sh.at[nd], o1.at[nd])

    return functools.partial(
        pl.kernel,
        out_type=(
            jax.ShapeDtypeStruct((n_pad, 16), jnp.float32),
            jax.ShapeDtypeStruct((n_pad, 16), jnp.float32),
        ),
        mesh=_mesh(),
        compiler_params=pltpu.CompilerParams(use_tc_tiling_on_sc=False),
        scratch_types=[
            pltpu.VMEM_SHARED((n_pad, 16), jnp.float32),
            pltpu.VMEM_SHARED((n_pad, 16), jnp.float32),
            pltpu.VMEM((KROW, LANE), jnp.int32),
            pltpu.VMEM((KROW, LANE), jnp.int32),
            pltpu.VMEM((LANE, 16), jnp.float32),
            pltpu.SemaphoreType.DMA,
        ],
        interpret=interpret,
    )(body_fn)


def _dis(cnt0_ref, cnt1_ref):
    return lax.rsqrt(cnt0_ref[:, :1] + cnt1_ref[:, :1] + 1.0)


def _stage_a(cnt0_ref, cnt1_ref, x_ref, w_ref, o_ref):
    dis = _dis(cnt0_ref, cnt1_ref)
    o_ref[...] = dis * jnp.dot(x_ref[...], w_ref[...],
                               preferred_element_type=jnp.float32)


def _stage_b1(s0, s1, hp, cnt0, cnt1, b, w, oa, ob):
    dis = _dis(cnt0, cnt1)
    act = jax.nn.relu(dis * (s0[...] + s1[...] + hp[...]) + b[...])
    h2 = dis * jnp.dot(act, w[...], preferred_element_type=jnp.float32)
    oa[...] = h2[:, :16]
    ob[...] = h2[:, 16:]


def _stage_b2(s2a, s2b, h2a, h2b, cnt0, cnt1, b, w, o):
    dis = _dis(cnt0, cnt1)
    s2 = jnp.concatenate([s2a[...], s2b[...]], axis=1)
    hp = jnp.concatenate([h2a[...], h2b[...]], axis=1)
    act = jax.nn.relu(dis * (s2 + hp) + b[...])
    o[...] = dis * jnp.dot(act, w[...], preferred_element_type=jnp.float32)


def _stage_b3(s0, s1, hp, cnt0, cnt1, b, w, o):
    dis = _dis(cnt0, cnt1)
    act = jax.nn.relu(dis * (s0[...] + s1[...] + hp[...]) + b[...])
    o[...] = dis * jnp.dot(act, w[...], preferred_element_type=jnp.float32)


def _stage_c(s0, s1, hp, cnt0, cnt1, b, o):
    dis = _dis(cnt0, cnt1)
    act = jax.nn.relu(dis * (s0[...] + s1[...] + hp[...]) + b[...])
    o[...] = act[:, :11]


def _tc_call(body, n_out, out_widths, ins, widths, n_rows, r):
    grid = (n_rows // r,)
    in_specs = []
    for a, w in zip(ins, widths):
        if w is None:
            in_specs.append(pl.BlockSpec(a.shape, lambda i, nd=a.ndim: (0,) * nd))
        else:
            in_specs.append(pl.BlockSpec((r, w), lambda i: (i, 0)))
    if n_out == 1:
        out_shape = jax.ShapeDtypeStruct((n_rows, out_widths[0]), jnp.float32)
        out_specs = pl.BlockSpec((r, out_widths[0]), lambda i: (i, 0))
    else:
        out_shape = tuple(jax.ShapeDtypeStruct((n_rows, w), jnp.float32)
                          for w in out_widths)
        out_specs = tuple(pl.BlockSpec((r, w), lambda i: (i, 0))
                          for w in out_widths)
    return pl.pallas_call(body, grid=grid, in_specs=in_specs,
                          out_specs=out_specs, out_shape=out_shape)(*ins)


def kernel(x, edge_index, W1, b1, W2, b2, W3, b3, W4, b4):
    f32 = jnp.float32
    src = edge_index[0].astype(jnp.int32)
    dst = edge_index[1].astype(jnp.int32)
    pad = jnp.full((E_PAD - N_EDGES,), N_NODES, jnp.int32)
    src2d = jnp.concatenate([src, pad]).reshape(E_ROWS, LANE)
    dst2d = jnp.concatenate([dst, pad]).reshape(E_ROWS, LANE)

    xpad = jnp.zeros((N_PAD, 16), f32).at[:N_NODES, :11].set(x)
    w1p = jnp.zeros((16, 16), f32).at[:11, :].set(W1)
    w4p = jnp.zeros((16, 16), f32).at[:, :11].set(W4)
    b1r = b1.reshape(1, 16)
    b2r = b2.reshape(1, 32)
    b3r = b3.reshape(1, 16)
    b4r = jnp.zeros((1, 16), f32).at[0, :11].set(b4)
    zeros_hbm = jnp.zeros((N_PAD, 16), f32)
    ones_hbm = jnp.ones((LANE, 16), f32)

    deg_k = _make_deg(N_PAD, E_ROWS)
    prop_e = _make_prop(N_PAD, E_ROWS, feat_split=False)
    prop_f = _make_prop(N_PAD, E_ROWS, feat_split=True)

    cnt0, cnt1 = deg_k(dst2d, zeros_hbm, ones_hbm)

    R = 1600
    hp1 = _tc_call(_stage_a, 1, (16,), (cnt0, cnt1, xpad, w1p),
                   (16, 16, 16, None), N_PAD, R)
    s10, s11 = prop_e(hp1, hp1, src2d, dst2d, zeros_hbm)
    h2a, h2b = _tc_call(_stage_b1, 2, (16, 16),
                        (s10, s11, hp1, cnt0, cnt1, b1r, W2),
                        (16, 16, 16, 16, 16, None, None), N_PAD, R)
    s2a, s2b = prop_f(h2a, h2b, src2d, dst2d, zeros_hbm)
    hp3 = _tc_call(_stage_b2, 1, (16,),
                   (s2a, s2b, h2a, h2b, cnt0, cnt1, b2r, W3),
                   (16, 16, 16, 16, 16, 16, None, None), N_PAD, R)
    s30, s31 = prop_e(hp3, hp3, src2d, dst2d, zeros_hbm)
    hp4 = _tc_call(_stage_b3, 1, (16,),
                   (s30, s31, hp3, cnt0, cnt1, b3r, w4p),
                   (16, 16, 16, 16, 16, None, None), N_PAD, R)
    s40, s41 = prop_e(hp4, hp4, src2d, dst2d, zeros_hbm)
    out = _tc_call(_stage_c, 1, (11,),
                   (s40, s41, hp4, cnt0, cnt1, b4r),
                   (16, 16, 16, 16, 16, None), N_NODES, 2000)
    return out


# speedup vs baseline: 77.6224x; 1.2256x over previous
import functools

import jax
import jax.numpy as jnp
from jax import lax
from jax.experimental import pallas as pl
from jax.experimental.pallas import tpu as pltpu
from jax.experimental.pallas import tpu_sc as plsc

N_NODES = 50000
N_PAD = 51200
N_EDGES = 3200000
LANE = 128
KROW = 8
E_PAD = 3211264
E_ROWS = E_PAD // LANE
NC, NS = 2, 16
CHUNK = KROW * LANE


def _mesh():
    return plsc.VectorSubcoreMesh(core_axis_name="c", subcore_axis_name="s",
                                  num_cores=NC, num_subcores=NS)


def _make_prop(n_pad, e_rows, feat_split):
    nworkers = NS if feat_split else NC * NS
    rows_per_tile = e_rows // nworkers
    nbody = rows_per_tile // (2 * KROW)
    rpt = n_pad // NS
    tbl_rows = (2 * n_pad) if feat_split else n_pad

    @functools.partial(
        pl.kernel,
        out_type=(
            jax.ShapeDtypeStruct((n_pad, 16), jnp.float32),
            jax.ShapeDtypeStruct((n_pad, 16), jnp.float32),
        ),
        mesh=_mesh(),
        compiler_params=pltpu.CompilerParams(use_tc_tiling_on_sc=False),
        scratch_types=[
            pltpu.VMEM_SHARED((n_pad, 16), jnp.float32),
            pltpu.VMEM((KROW, LANE), jnp.int32),
            pltpu.VMEM((KROW, LANE), jnp.int32),
            pltpu.VMEM((KROW, LANE), jnp.int32),
            pltpu.VMEM((KROW, LANE), jnp.int32),
            pltpu.VMEM((CHUNK, 16), jnp.float32),
            pltpu.VMEM((CHUNK, 16), jnp.float32),
            pltpu.SemaphoreType.DMA,
            pltpu.SemaphoreType.DMA,
            pltpu.SemaphoreType.DMA,
            pltpu.SemaphoreType.DMA,
        ],
    )
    def prop_kernel(tbl_hbm, srca_hbm, srcb_hbm, dst_hbm, zeros_hbm, o0, o1,
                    acc_sh, sxa, sxb, dxa, dxb, rowsa, rowsb,
                    sga, sgb, ssa, ssb):
        cid = lax.axis_index("c")
        sid = lax.axis_index("s")
        nd = pl.ds(sid * rpt, rpt)
        pltpu.sync_copy(zeros_hbm.at[nd], acc_sh.at[nd])
        plsc.subcore_barrier()

        if feat_split:
            base = sid * rows_per_tile
        else:
            base = (cid * NS + sid) * rows_per_tile

        def body(t, carry):
            row = base + t * 2 * KROW

            def bank(r0, sx, dx, rows, sg, ss):
                @pl.when(cid == 0)
                def _():
                    pltpu.sync_copy(srca_hbm.at[pl.ds(r0, KROW)], sx)

                @pl.when(cid == 1)
                def _():
                    pltpu.sync_copy(srcb_hbm.at[pl.ds(r0, KROW)], sx)
                pltpu.sync_copy(dst_hbm.at[pl.ds(r0, KROW)], dx)
                return [
                    pltpu.async_copy(tbl_hbm.at[sx.at[j]],
                                     rows.at[pl.ds(j * LANE, LANE)], sg)
                    for j in range(KROW)
                ]

            def scatter(sx, dx, rows, gd, ss):
                for d in gd:
                    d.wait()
                return [
                    pltpu.async_copy(rows.at[pl.ds(j * LANE, LANE)],
                                     acc_sh.at[dx.at[j]], ss, add=True)
                    for j in range(KROW)
                ]

            gda = bank(row, sxa, dxa, rowsa, sga, ssa)
            gdb = bank(row + KROW, sxb, dxb, rowsb, sgb, ssb)
            sda = scatter(sxa, dxa, rowsa, gda, ssa)
            sdb = scatter(sxb, dxb, rowsb, gdb, ssb)
            for d in sda:
                d.wait()
            for d in sdb:
                d.wait()
            return carry

        lax.fori_loop(0, nbody, body, 0)
        plsc.subcore_barrier()

        @pl.when(cid == 0)
        def _():
            pltpu.sync_copy(acc_sh.at[nd], o0.at[nd])

        @pl.when(cid == 1)
        def _():
            pltpu.sync_copy(acc_sh.at[nd], o1.at[nd])

    return prop_kernel


def _make_deg(n_pad, e_rows):
    rows_per_tile = e_rows // (NC * NS)
    nbody = rows_per_tile // (2 * KROW)
    rpt = n_pad // NS

    @functools.partial(
        pl.kernel,
        out_type=(
            jax.ShapeDtypeStruct((n_pad, 16), jnp.float32),
            jax.ShapeDtypeStruct((n_pad, 16), jnp.float32),
        ),
        mesh=_mesh(),
        compiler_params=pltpu.CompilerParams(use_tc_tiling_on_sc=False),
        scratch_types=[
            pltpu.VMEM_SHARED((n_pad, 16), jnp.float32),
            pltpu.VMEM((KROW, LANE), jnp.int32),
            pltpu.VMEM((KROW, LANE), jnp.int32),
            pltpu.VMEM((LANE, 16), jnp.float32),
            pltpu.SemaphoreType.DMA,
            pltpu.SemaphoreType.DMA,
        ],
    )
    def deg_kernel(dst_hbm, zeros_hbm, ones_hbm, o0, o1,
                   acc_sh, dxa, dxb, ones_v, ssa, ssb):
        cid = lax.axis_index("c")
        sid = lax.axis_index("s")
        nd = pl.ds(sid * rpt, rpt)
        pltpu.sync_copy(zeros_hbm.at[nd], acc_sh.at[nd])
        pltpu.sync_copy(ones_hbm, ones_v)
        plsc.subcore_barrier()
        base = (cid * NS + sid) * rows_per_tile

        def body(t, carry):
            row = base + t * 2 * KROW

            def bank(r0, dx, ss):
                pltpu.sync_copy(dst_hbm.at[pl.ds(r0, KROW)], dx)
                return [
                    pltpu.async_copy(ones_v, acc_sh.at[dx.at[j]], ss, add=True)
                    for j in range(KROW)
                ]

            sda = bank(row, dxa, ssa)
            sdb = bank(row + KROW, dxb, ssb)
            for d in sda:
                d.wait()
            for d in sdb:
                d.wait()
            return carry

        lax.fori_loop(0, nbody, body, 0)
        plsc.subcore_barrier()

        @pl.when(cid == 0)
        def _():
            pltpu.sync_copy(acc_sh.at[nd], o0.at[nd])

        @pl.when(cid == 1)
        def _():
            pltpu.sync_copy(acc_sh.at[nd], o1.at[nd])

    return deg_kernel


def _dis(cnt0_ref, cnt1_ref):
    return lax.rsqrt(cnt0_ref[:, :1] + cnt1_ref[:, :1] + 1.0)


def _stage_a(cnt0_ref, cnt1_ref, x_ref, w_ref, o_ref):
    dis = _dis(cnt0_ref, cnt1_ref)
    o_ref[...] = dis * jnp.dot(x_ref[...], w_ref[...],
                               preferred_element_type=jnp.float32)


def _stage_b(s0, s1, hp, cnt0, cnt1, b, w, o):
    dis = _dis(cnt0, cnt1)
    act = jax.nn.relu(dis * (s0[...] + s1[...] + hp[...]) + b[...])
    o[...] = dis * jnp.dot(act, w[...], preferred_element_type=jnp.float32)


def _stage_b1_body(s0, s1, hp, cnt0, cnt1, b, w, oa, ob):
    dis = _dis(cnt0, cnt1)
    act = jax.nn.relu(dis * (s0[...] + s1[...] + hp[...]) + b[...])
    h2 = dis * jnp.dot(act, w[...], preferred_element_type=jnp.float32)
    oa[...] = h2[:, :16]
    ob[...] = h2[:, 16:]


def _tc_call_b1(ins, n_rows, r):
    grid = (n_rows // r,)
    widths = (16, 16, 16, 16, 16, None, None)
    in_specs = []
    for a, w in zip(ins, widths):
        if w is None:
            in_specs.append(pl.BlockSpec(a.shape, lambda i, nd=a.ndim: (0,) * nd))
        else:
            in_specs.append(pl.BlockSpec((r, w), lambda i: (i, 0)))
    out_shape = tuple(jax.ShapeDtypeStruct((n_rows, 16), jnp.float32) for _ in range(2))
    out_specs = tuple(pl.BlockSpec((r, 16), lambda i: (i, 0)) for _ in range(2))
    return pl.pallas_call(_stage_b1_body, grid=grid, in_specs=in_specs,
                          out_specs=out_specs, out_shape=out_shape)(*ins)


def _stage_b2(s2a, s2b, h2a, h2b, cnt0, cnt1, b, w, o):
    dis = _dis(cnt0, cnt1)
    s2 = jnp.concatenate([s2a[...], s2b[...]], axis=1)
    hp = jnp.concatenate([h2a[...], h2b[...]], axis=1)
    act = jax.nn.relu(dis * (s2 + hp) + b[...])
    o[...] = dis * jnp.dot(act, w[...], preferred_element_type=jnp.float32)


def _stage_c(s0, s1, hp, cnt0, cnt1, b, o):
    dis = _dis(cnt0, cnt1)
    act = jax.nn.relu(dis * (s0[...] + s1[...] + hp[...]) + b[...])
    o[...] = act[:, :11]


def _tc_call(body, out_width, ins, widths, n_rows, r):
    grid = (n_rows // r,)
    in_specs = []
    for a, w in zip(ins, widths):
        if w is None:
            in_specs.append(pl.BlockSpec(a.shape, lambda i, nd=a.ndim: (0,) * nd))
        else:
            in_specs.append(pl.BlockSpec((r, w), lambda i: (i, 0)))
    out_shape = jax.ShapeDtypeStruct((n_rows, out_width), jnp.float32)
    out_specs = pl.BlockSpec((r, out_width), lambda i: (i, 0))
    return pl.pallas_call(body, grid=grid, in_specs=in_specs,
                          out_specs=out_specs, out_shape=out_shape)(*ins)


def kernel(x, edge_index, W1, b1, W2, b2, W3, b3, W4, b4):
    f32 = jnp.float32
    src = edge_index[0].astype(jnp.int32)
    dst = edge_index[1].astype(jnp.int32)
    pad = jnp.full((E_PAD - N_EDGES,), N_NODES, jnp.int32)
    src2d = jnp.concatenate([src, pad]).reshape(E_ROWS, LANE)
    dst2d = jnp.concatenate([dst, pad]).reshape(E_ROWS, LANE)

    xpad = jnp.zeros((N_PAD, 16), f32).at[:N_NODES, :11].set(x)
    w1p = jnp.zeros((16, 16), f32).at[:11, :].set(W1)
    w4p = jnp.zeros((16, 16), f32).at[:, :11].set(W4)
    b1r = b1.reshape(1, 16)
    b2r = b2.reshape(1, 32)
    b3r = b3.reshape(1, 16)
    b4r = jnp.zeros((1, 16), f32).at[0, :11].set(b4)
    zeros16 = jnp.zeros((N_PAD, 16), f32)
    zeros32 = jnp.zeros((N_PAD, 32), f32)
    ones_hbm = jnp.ones((LANE, 16), f32)

    deg_k = _make_deg(N_PAD, E_ROWS)
    prop_e = _make_prop(N_PAD, E_ROWS, feat_split=False)
    prop_f = _make_prop(N_PAD, E_ROWS, feat_split=True)

    srcoff2d = src2d + N_PAD

    cnt0, cnt1 = deg_k(dst2d, zeros16, ones_hbm)

    R = 1600
    hp1 = _tc_call(_stage_a, 16, (cnt0, cnt1, xpad, w1p),
                   (16, 16, 16, None), N_PAD, R)
    s10, s11 = prop_e(hp1, src2d, src2d, dst2d, zeros16)
    hp2a, hp2b = _tc_call_b1((s10, s11, hp1, cnt0, cnt1, b1r, W2), N_PAD, R)
    tbl2 = jnp.concatenate([hp2a, hp2b], axis=0)
    s2a, s2b = prop_f(tbl2, src2d, srcoff2d, dst2d, zeros16)
    hp3 = _tc_call(_stage_b2, 16, (s2a, s2b, hp2a, hp2b, cnt0, cnt1, b2r, W3),
                   (16, 16, 16, 16, 16, 16, None, None), N_PAD, R)
    s30, s31 = prop_e(hp3, src2d, src2d, dst2d, zeros16)
    hp4 = _tc_call(_stage_b, 16, (s30, s31, hp3, cnt0, cnt1, b3r, w4p),
                   (16, 16, 16, 16, 16, None, None), N_PAD, R)
    s40, s41 = prop_e(hp4, src2d, src2d, dst2d, zeros16)
    out = _tc_call(_stage_c, 11, (s40, s41, hp4, cnt0, cnt1, b4r),
                   (16, 16, 16, 16, 16, None), N_NODES, 2000)
    return out


# speedup vs baseline: 79.4882x; 1.0240x over previous
import functools

import jax
import jax.numpy as jnp
from jax import lax
from jax.experimental import pallas as pl
from jax.experimental.pallas import tpu as pltpu
from jax.experimental.pallas import tpu_sc as plsc

N_NODES = 50000
N_PAD = 51200
N_EDGES = 3200000
LANE = 128
KROW = 8
E_PAD = 3211264
E_ROWS = E_PAD // LANE
NC, NS = 2, 16
CH = 1024


def _mesh():
    return plsc.VectorSubcoreMesh(core_axis_name="c", subcore_axis_name="s",
                                  num_cores=NC, num_subcores=NS)


def _make_prop(n_pad, e_pad, feat_split):
    nworkers = NS if feat_split else NC * NS
    edges_per_tile = e_pad // nworkers
    nbody = edges_per_tile // (2 * CH)
    rpt = n_pad // NS

    @functools.partial(
        pl.kernel,
        out_type=(
            jax.ShapeDtypeStruct((n_pad, 16), jnp.float32),
            jax.ShapeDtypeStruct((n_pad, 16), jnp.float32),
        ),
        mesh=_mesh(),
        compiler_params=pltpu.CompilerParams(use_tc_tiling_on_sc=False),
        scratch_types=[
            pltpu.VMEM_SHARED((n_pad, 16), jnp.float32),
            pltpu.VMEM((CH,), jnp.int32),
            pltpu.VMEM((CH,), jnp.int32),
            pltpu.VMEM((CH,), jnp.int32),
            pltpu.VMEM((CH,), jnp.int32),
            pltpu.VMEM((CH, 16), jnp.float32),
            pltpu.VMEM((CH, 16), jnp.float32),
            pltpu.SemaphoreType.DMA,
            pltpu.SemaphoreType.DMA,
            pltpu.SemaphoreType.DMA,
            pltpu.SemaphoreType.DMA,
        ],
    )
    def prop_kernel(tbl_hbm, srca_hbm, srcb_hbm, dst_hbm, zeros_hbm, o0, o1,
                    acc_sh, sxa, sxb, dxa, dxb, rowsa, rowsb,
                    sga, sgb, ssa, ssb):
        cid = lax.axis_index("c")
        sid = lax.axis_index("s")
        nd = pl.ds(sid * rpt, rpt)
        pltpu.sync_copy(zeros_hbm.at[nd], acc_sh.at[nd])
        plsc.subcore_barrier()

        if feat_split:
            base = sid * edges_per_tile
        else:
            base = (cid * NS + sid) * edges_per_tile

        def body(t, carry):
            e0 = base + t * 2 * CH

            def bank(eo, sx, dx, rows, sg):
                @pl.when(cid == 0)
                def _():
                    pltpu.sync_copy(srca_hbm.at[pl.ds(eo, CH)], sx)

                @pl.when(cid == 1)
                def _():
                    pltpu.sync_copy(srcb_hbm.at[pl.ds(eo, CH)], sx)
                pltpu.sync_copy(dst_hbm.at[pl.ds(eo, CH)], dx)
                return pltpu.async_copy(tbl_hbm.at[sx], rows, sg)

            gda = bank(e0, sxa, dxa, rowsa, sga)
            gdb = bank(e0 + CH, sxb, dxb, rowsb, sgb)
            gda.wait()
            sda = pltpu.async_copy(rowsa, acc_sh.at[dxa], ssa, add=True)
            gdb.wait()
            sdb = pltpu.async_copy(rowsb, acc_sh.at[dxb], ssb, add=True)
            sda.wait()
            sdb.wait()
            return carry

        lax.fori_loop(0, nbody, body, 0)
        plsc.subcore_barrier()

        @pl.when(cid == 0)
        def _():
            pltpu.sync_copy(acc_sh.at[nd], o0.at[nd])

        @pl.when(cid == 1)
        def _():
            pltpu.sync_copy(acc_sh.at[nd], o1.at[nd])

    return prop_kernel


def _make_deg(n_pad, e_pad):
    edges_per_tile = e_pad // (NC * NS)
    nbody = edges_per_tile // (2 * CH)
    rpt = n_pad // NS

    @functools.partial(
        pl.kernel,
        out_type=(
            jax.ShapeDtypeStruct((n_pad, 16), jnp.float32),
            jax.ShapeDtypeStruct((n_pad, 16), jnp.float32),
        ),
        mesh=_mesh(),
        compiler_params=pltpu.CompilerParams(use_tc_tiling_on_sc=False),
        scratch_types=[
            pltpu.VMEM_SHARED((n_pad, 16), jnp.float32),
            pltpu.VMEM((CH,), jnp.int32),
            pltpu.VMEM((CH,), jnp.int32),
            pltpu.VMEM((CH, 16), jnp.float32),
            pltpu.SemaphoreType.DMA,
            pltpu.SemaphoreType.DMA,
        ],
    )
    def deg_kernel(dst_hbm, zeros_hbm, ones_hbm, o0, o1,
                   acc_sh, dxa, dxb, ones_v, ssa, ssb):
        cid = lax.axis_index("c")
        sid = lax.axis_index("s")
        nd = pl.ds(sid * rpt, rpt)
        pltpu.sync_copy(zeros_hbm.at[nd], acc_sh.at[nd])
        pltpu.sync_copy(ones_hbm, ones_v)
        plsc.subcore_barrier()
        base = (cid * NS + sid) * edges_per_tile

        def body(t, carry):
            e0 = base + t * 2 * CH
            pltpu.sync_copy(dst_hbm.at[pl.ds(e0, CH)], dxa)
            sda = pltpu.async_copy(ones_v, acc_sh.at[dxa], ssa, add=True)
            pltpu.sync_copy(dst_hbm.at[pl.ds(e0 + CH, CH)], dxb)
            sdb = pltpu.async_copy(ones_v, acc_sh.at[dxb], ssb, add=True)
            sda.wait()
            sdb.wait()
            return carry

        lax.fori_loop(0, nbody, body, 0)
        plsc.subcore_barrier()

        @pl.when(cid == 0)
        def _():
            pltpu.sync_copy(acc_sh.at[nd], o0.at[nd])

        @pl.when(cid == 1)
        def _():
            pltpu.sync_copy(acc_sh.at[nd], o1.at[nd])

    return deg_kernel


def _dis(cnt0_ref, cnt1_ref):
    return lax.rsqrt(cnt0_ref[:, :1] + cnt1_ref[:, :1] + 1.0)


def _stage_a(cnt0_ref, cnt1_ref, x_ref, w_ref, o_ref):
    dis = _dis(cnt0_ref, cnt1_ref)
    o_ref[...] = dis * jnp.dot(x_ref[...], w_ref[...],
                               preferred_element_type=jnp.float32)


def _stage_b(s0, s1, hp, cnt0, cnt1, b, w, o):
    dis = _dis(cnt0, cnt1)
    act = jax.nn.relu(dis * (s0[...] + s1[...] + hp[...]) + b[...])
    o[...] = dis * jnp.dot(act, w[...], preferred_element_type=jnp.float32)


def _stage_b1_body(s0, s1, hp, cnt0, cnt1, b, w, oa, ob):
    dis = _dis(cnt0, cnt1)
    act = jax.nn.relu(dis * (s0[...] + s1[...] + hp[...]) + b[...])
    h2 = dis * jnp.dot(act, w[...], preferred_element_type=jnp.float32)
    oa[...] = h2[:, :16]
    ob[...] = h2[:, 16:]


def _tc_call_b1(ins, n_rows, r):
    grid = (n_rows // r,)
    widths = (16, 16, 16, 16, 16, None, None)
    in_specs = []
    for a, w in zip(ins, widths):
        if w is None:
            in_specs.append(pl.BlockSpec(a.shape, lambda i, nd=a.ndim: (0,) * nd))
        else:
            in_specs.append(pl.BlockSpec((r, w), lambda i: (i, 0)))
    out_shape = tuple(jax.ShapeDtypeStruct((n_rows, 16), jnp.float32) for _ in range(2))
    out_specs = tuple(pl.BlockSpec((r, 16), lambda i: (i, 0)) for _ in range(2))
    return pl.pallas_call(_stage_b1_body, grid=grid, in_specs=in_specs,
                          out_specs=out_specs, out_shape=out_shape)(*ins)


def _stage_b2(s2a, s2b, h2a, h2b, cnt0, cnt1, b, w, o):
    dis = _dis(cnt0, cnt1)
    s2 = jnp.concatenate([s2a[...], s2b[...]], axis=1)
    hp = jnp.concatenate([h2a[...], h2b[...]], axis=1)
    act = jax.nn.relu(dis * (s2 + hp) + b[...])
    o[...] = dis * jnp.dot(act, w[...], preferred_element_type=jnp.float32)


def _stage_c(s0, s1, hp, cnt0, cnt1, b, o):
    dis = _dis(cnt0, cnt1)
    act = jax.nn.relu(dis * (s0[...] + s1[...] + hp[...]) + b[...])
    o[...] = act[:, :11]


def _tc_call(body, out_width, ins, widths, n_rows, r):
    grid = (n_rows // r,)
    in_specs = []
    for a, w in zip(ins, widths):
        if w is None:
            in_specs.append(pl.BlockSpec(a.shape, lambda i, nd=a.ndim: (0,) * nd))
        else:
            in_specs.append(pl.BlockSpec((r, w), lambda i: (i, 0)))
    out_shape = jax.ShapeDtypeStruct((n_rows, out_width), jnp.float32)
    out_specs = pl.BlockSpec((r, out_width), lambda i: (i, 0))
    return pl.pallas_call(body, grid=grid, in_specs=in_specs,
                          out_specs=out_specs, out_shape=out_shape)(*ins)


def kernel(x, edge_index, W1, b1, W2, b2, W3, b3, W4, b4):
    f32 = jnp.float32
    src = edge_index[0].astype(jnp.int32)
    dst = edge_index[1].astype(jnp.int32)
    pad = jnp.full((E_PAD - N_EDGES,), N_NODES, jnp.int32)
    src1 = jnp.concatenate([src, pad])
    dst1 = jnp.concatenate([dst, pad])

    xpad = jnp.zeros((N_PAD, 16), f32).at[:N_NODES, :11].set(x)
    w1p = jnp.zeros((16, 16), f32).at[:11, :].set(W1)
    w4p = jnp.zeros((16, 16), f32).at[:, :11].set(W4)
    b1r = b1.reshape(1, 16)
    b2r = b2.reshape(1, 32)
    b3r = b3.reshape(1, 16)
    b4r = jnp.zeros((1, 16), f32).at[0, :11].set(b4)
    zeros16 = jnp.zeros((N_PAD, 16), f32)
    zeros32 = jnp.zeros((N_PAD, 32), f32)
    zeros32 = jnp.zeros((N_PAD, 32), f32)
    ones_hbm = jnp.ones((CH, 16), f32)

    deg_k = _make_deg(N_PAD, E_PAD)
    prop_e = _make_prop(N_PAD, E_PAD, feat_split=False)
    prop_f = _make_prop(N_PAD, E_PAD, feat_split=True)

    srcoff1 = src1 + N_PAD

    cnt0, cnt1 = deg_k(dst1, zeros16, ones_hbm)

    R = 1600
    hp1 = _tc_call(_stage_a, 16, (cnt0, cnt1, xpad, w1p),
                   (16, 16, 16, None), N_PAD, R)
    s10, s11 = prop_e(hp1, src1, src1, dst1, zeros16)
    hp2a, hp2b = _tc_call_b1((s10, s11, hp1, cnt0, cnt1, b1r, W2), N_PAD, R)
    tbl2 = jnp.concatenate([hp2a, hp2b], axis=0)
    s2a, s2b = prop_f(tbl2, src1, srcoff1, dst1, zeros16)
    hp3 = _tc_call(_stage_b2, 16, (s2a, s2b, hp2a, hp2b, cnt0, cnt1, b2r, W3),
                   (16, 16, 16, 16, 16, 16, None, None), N_PAD, R)
    s30, s31 = prop_e(hp3, src1, src1, dst1, zeros16)
    hp4 = _tc_call(_stage_b, 16, (s30, s31, hp3, cnt0, cnt1, b3r, w4p),
                   (16, 16, 16, 16, 16, None, None), N_PAD, R)
    s40, s41 = prop_e(hp4, src1, src1, dst1, zeros16)
    out = _tc_call(_stage_c, 11, (s40, s41, hp4, cnt0, cnt1, b4r),
                   (16, 16, 16, 16, 16, None), N_NODES, 2000)
    return out


# speedup vs baseline: 93.6938x; 1.1787x over previous
import functools

import jax
import jax.numpy as jnp
from jax import lax
from jax.experimental import pallas as pl
from jax.experimental.pallas import tpu as pltpu
from jax.experimental.pallas import tpu_sc as plsc

N_NODES = 50000
N_PAD = 51200
N_EDGES = 3200000
LANE = 128
KROW = 8
E_PAD = 3211264
E_ROWS = E_PAD // LANE
NC, NS = 2, 16
CH = 2048


def _mesh():
    return plsc.VectorSubcoreMesh(core_axis_name="c", subcore_axis_name="s",
                                  num_cores=NC, num_subcores=NS)


def _make_prop(n_pad, e_pad, feat_split):
    nworkers = NS if feat_split else NC * NS
    edges_per_tile = e_pad // nworkers
    nbody = edges_per_tile // (2 * CH)
    rpt = n_pad // NS

    @functools.partial(
        pl.kernel,
        out_type=(
            jax.ShapeDtypeStruct((n_pad, 16), jnp.float32),
            jax.ShapeDtypeStruct((n_pad, 16), jnp.float32),
        ),
        mesh=_mesh(),
        compiler_params=pltpu.CompilerParams(use_tc_tiling_on_sc=False),
        scratch_types=[
            pltpu.VMEM_SHARED((n_pad, 16), jnp.float32),
            pltpu.VMEM((CH,), jnp.int32),
            pltpu.VMEM((CH,), jnp.int32),
            pltpu.VMEM((CH,), jnp.int32),
            pltpu.VMEM((CH,), jnp.int32),
            pltpu.VMEM((CH, 16), jnp.float32),
            pltpu.VMEM((CH, 16), jnp.float32),
            pltpu.SemaphoreType.DMA,
            pltpu.SemaphoreType.DMA,
            pltpu.SemaphoreType.DMA,
            pltpu.SemaphoreType.DMA,
        ],
    )
    def prop_kernel(tbl_hbm, srca_hbm, srcb_hbm, dst_hbm, zeros_hbm, o0, o1,
                    acc_sh, sxa, sxb, dxa, dxb, rowsa, rowsb,
                    sga, sgb, ssa, ssb):
        cid = lax.axis_index("c")
        sid = lax.axis_index("s")
        nd = pl.ds(sid * rpt, rpt)
        pltpu.sync_copy(zeros_hbm.at[nd], acc_sh.at[nd])
        plsc.subcore_barrier()

        if feat_split:
            base = sid * edges_per_tile
        else:
            base = (cid * NS + sid) * edges_per_tile

        def body(t, carry):
            e0 = base + t * 2 * CH

            def bank(eo, sx, dx, rows, sg):
                @pl.when(cid == 0)
                def _():
                    pltpu.sync_copy(srca_hbm.at[pl.ds(eo, CH)], sx)

                @pl.when(cid == 1)
                def _():
                    pltpu.sync_copy(srcb_hbm.at[pl.ds(eo, CH)], sx)
                pltpu.sync_copy(dst_hbm.at[pl.ds(eo, CH)], dx)
                return pltpu.async_copy(tbl_hbm.at[sx], rows, sg)

            gda = bank(e0, sxa, dxa, rowsa, sga)
            gdb = bank(e0 + CH, sxb, dxb, rowsb, sgb)
            gda.wait()
            sda = pltpu.async_copy(rowsa, acc_sh.at[dxa], ssa, add=True)
            gdb.wait()
            sdb = pltpu.async_copy(rowsb, acc_sh.at[dxb], ssb, add=True)
            sda.wait()
            sdb.wait()
            return carry

        lax.fori_loop(0, nbody, body, 0)
        plsc.subcore_barrier()

        @pl.when(cid == 0)
        def _():
            pltpu.sync_copy(acc_sh.at[nd], o0.at[nd])

        @pl.when(cid == 1)
        def _():
            pltpu.sync_copy(acc_sh.at[nd], o1.at[nd])

    return prop_kernel


def _make_deg(n_pad, e_pad):
    edges_per_tile = e_pad // (NC * NS)
    nbody = edges_per_tile // (2 * CH)
    rpt = n_pad // NS

    @functools.partial(
        pl.kernel,
        out_type=(
            jax.ShapeDtypeStruct((n_pad, 16), jnp.float32),
            jax.ShapeDtypeStruct((n_pad, 16), jnp.float32),
        ),
        mesh=_mesh(),
        compiler_params=pltpu.CompilerParams(use_tc_tiling_on_sc=False),
        scratch_types=[
            pltpu.VMEM_SHARED((n_pad, 16), jnp.float32),
            pltpu.VMEM((CH,), jnp.int32),
            pltpu.VMEM((CH,), jnp.int32),
            pltpu.VMEM((CH, 16), jnp.float32),
            pltpu.SemaphoreType.DMA,
            pltpu.SemaphoreType.DMA,
        ],
    )
    def deg_kernel(dst_hbm, zeros_hbm, ones_hbm, o0, o1,
                   acc_sh, dxa, dxb, ones_v, ssa, ssb):
        cid = lax.axis_index("c")
        sid = lax.axis_index("s")
        nd = pl.ds(sid * rpt, rpt)
        pltpu.sync_copy(zeros_hbm.at[nd], acc_sh.at[nd])
        pltpu.sync_copy(ones_hbm, ones_v)
        plsc.subcore_barrier()
        base = (cid * NS + sid) * edges_per_tile

        def body(t, carry):
            e0 = base + t * 2 * CH
            pltpu.sync_copy(dst_hbm.at[pl.ds(e0, CH)], dxa)
            sda = pltpu.async_copy(ones_v, acc_sh.at[dxa], ssa, add=True)
            pltpu.sync_copy(dst_hbm.at[pl.ds(e0 + CH, CH)], dxb)
            sdb = pltpu.async_copy(ones_v, acc_sh.at[dxb], ssb, add=True)
            sda.wait()
            sdb.wait()
            return carry

        lax.fori_loop(0, nbody, body, 0)
        plsc.subcore_barrier()

        @pl.when(cid == 0)
        def _():
            pltpu.sync_copy(acc_sh.at[nd], o0.at[nd])

        @pl.when(cid == 1)
        def _():
            pltpu.sync_copy(acc_sh.at[nd], o1.at[nd])

    return deg_kernel


def _dis(cnt0_ref, cnt1_ref):
    return lax.rsqrt(cnt0_ref[:, :1] + cnt1_ref[:, :1] + 1.0)


def _stage_a(cnt0_ref, cnt1_ref, x_ref, w_ref, o_ref):
    dis = _dis(cnt0_ref, cnt1_ref)
    o_ref[...] = dis * jnp.dot(x_ref[...], w_ref[...],
                               preferred_element_type=jnp.float32)


def _stage_b(s0, s1, hp, cnt0, cnt1, b, w, o):
    dis = _dis(cnt0, cnt1)
    act = jax.nn.relu(dis * (s0[...] + s1[...] + hp[...]) + b[...])
    o[...] = dis * jnp.dot(act, w[...], preferred_element_type=jnp.float32)


def _stage_b1_body(s0, s1, hp, cnt0, cnt1, b, w, oa, ob):
    dis = _dis(cnt0, cnt1)
    act = jax.nn.relu(dis * (s0[...] + s1[...] + hp[...]) + b[...])
    h2 = dis * jnp.dot(act, w[...], preferred_element_type=jnp.float32)
    oa[...] = h2[:, :16]
    ob[...] = h2[:, 16:]


def _tc_call_b1(ins, n_rows, r):
    grid = (n_rows // r,)
    widths = (16, 16, 16, 16, 16, None, None)
    in_specs = []
    for a, w in zip(ins, widths):
        if w is None:
            in_specs.append(pl.BlockSpec(a.shape, lambda i, nd=a.ndim: (0,) * nd))
        else:
            in_specs.append(pl.BlockSpec((r, w), lambda i: (i, 0)))
    out_shape = tuple(jax.ShapeDtypeStruct((n_rows, 16), jnp.float32) for _ in range(2))
    out_specs = tuple(pl.BlockSpec((r, 16), lambda i: (i, 0)) for _ in range(2))
    return pl.pallas_call(_stage_b1_body, grid=grid, in_specs=in_specs,
                          out_specs=out_specs, out_shape=out_shape)(*ins)


def _stage_b2(s2a, s2b, h2a, h2b, cnt0, cnt1, b, w, o):
    dis = _dis(cnt0, cnt1)
    s2 = jnp.concatenate([s2a[...], s2b[...]], axis=1)
    hp = jnp.concatenate([h2a[...], h2b[...]], axis=1)
    act = jax.nn.relu(dis * (s2 + hp) + b[...])
    o[...] = dis * jnp.dot(act, w[...], preferred_element_type=jnp.float32)


def _stage_c(s0, s1, hp, cnt0, cnt1, b, o):
    dis = _dis(cnt0, cnt1)
    act = jax.nn.relu(dis * (s0[...] + s1[...] + hp[...]) + b[...])
    o[...] = act[:, :11]


def _tc_call(body, out_width, ins, widths, n_rows, r):
    grid = (n_rows // r,)
    in_specs = []
    for a, w in zip(ins, widths):
        if w is None:
            in_specs.append(pl.BlockSpec(a.shape, lambda i, nd=a.ndim: (0,) * nd))
        else:
            in_specs.append(pl.BlockSpec((r, w), lambda i: (i, 0)))
    out_shape = jax.ShapeDtypeStruct((n_rows, out_width), jnp.float32)
    out_specs = pl.BlockSpec((r, out_width), lambda i: (i, 0))
    return pl.pallas_call(body, grid=grid, in_specs=in_specs,
                          out_specs=out_specs, out_shape=out_shape)(*ins)


def kernel(x, edge_index, W1, b1, W2, b2, W3, b3, W4, b4):
    f32 = jnp.float32
    src = edge_index[0].astype(jnp.int32)
    dst = edge_index[1].astype(jnp.int32)
    pad = jnp.full((E_PAD - N_EDGES,), N_NODES, jnp.int32)
    src1 = jnp.concatenate([src, pad])
    dst1 = jnp.concatenate([dst, pad])

    xpad = jnp.zeros((N_PAD, 16), f32).at[:N_NODES, :11].set(x)
    w1p = jnp.zeros((16, 16), f32).at[:11, :].set(W1)
    w4p = jnp.zeros((16, 16), f32).at[:, :11].set(W4)
    b1r = b1.reshape(1, 16)
    b2r = b2.reshape(1, 32)
    b3r = b3.reshape(1, 16)
    b4r = jnp.zeros((1, 16), f32).at[0, :11].set(b4)
    zeros16 = jnp.zeros((N_PAD, 16), f32)
    zeros32 = jnp.zeros((N_PAD, 32), f32)
    zeros32 = jnp.zeros((N_PAD, 32), f32)
    ones_hbm = jnp.ones((CH, 16), f32)

    deg_k = _make_deg(N_PAD, E_PAD)
    prop_e = _make_prop(N_PAD, E_PAD, feat_split=False)
    prop_f = _make_prop(N_PAD, E_PAD, feat_split=True)

    srcoff1 = src1 + N_PAD

    cnt0, cnt1 = deg_k(dst1, zeros16, ones_hbm)

    R = 1600
    hp1 = _tc_call(_stage_a, 16, (cnt0, cnt1, xpad, w1p),
                   (16, 16, 16, None), N_PAD, R)
    s10, s11 = prop_e(hp1, src1, src1, dst1, zeros16)
    hp2a, hp2b = _tc_call_b1((s10, s11, hp1, cnt0, cnt1, b1r, W2), N_PAD, R)
    tbl2 = jnp.concatenate([hp2a, hp2b], axis=0)
    s2a, s2b = prop_f(tbl2, src1, srcoff1, dst1, zeros16)
    hp3 = _tc_call(_stage_b2, 16, (s2a, s2b, hp2a, hp2b, cnt0, cnt1, b2r, W3),
                   (16, 16, 16, 16, 16, 16, None, None), N_PAD, R)
    s30, s31 = prop_e(hp3, src1, src1, dst1, zeros16)
    hp4 = _tc_call(_stage_b, 16, (s30, s31, hp3, cnt0, cnt1, b3r, w4p),
                   (16, 16, 16, 16, 16, None, None), N_PAD, R)
    s40, s41 = prop_e(hp4, src1, src1, dst1, zeros16)
    out = _tc_call(_stage_c, 11, (s40, s41, hp4, cnt0, cnt1, b4r),
                   (16, 16, 16, 16, 16, None), N_NODES, 2000)
    return out
